# baseline (speedup 1.0000x reference)
import functools

import jax
import jax.numpy as jnp
from jax import lax
from jax.experimental import pallas as pl
from jax.experimental.pallas import tpu as pltpu
from jax.experimental.pallas import tpu_sc as plsc

N = 10000
E = 320000
D = 128
B = 100
NN = 100
DS = 64
DH = 64

NC, NS = 2, 16
NW = NC * NS
EW = E // NW

NPAD = 10240
RPS = NPAD // NS
ZR = 64

NCH = 80
CH = 128
EWP = NCH * CH

_f32 = jnp.float32


def _make_sc_agg(with_deg, preload_dst):
    mesh = plsc.VectorSubcoreMesh(core_axis_name="c", subcore_axis_name="s")

    out_type = [jax.ShapeDtypeStruct((NC * NPAD, D), _f32)]
    scratch = [
        pltpu.VMEM((CH,), jnp.int32),
        pltpu.VMEM((CH,), jnp.int32),
        pltpu.VMEM((CH, D), _f32),
        pltpu.VMEM((CH, D), _f32),
        pltpu.VMEM_SHARED((NPAD, D), _f32),
        pltpu.SemaphoreType.DMA,
        pltpu.SemaphoreType.DMA,
        pltpu.SemaphoreType.DMA,
        pltpu.SemaphoreType.DMA,
        pltpu.SemaphoreType.DMA,
    ]
    if preload_dst:
        scratch.insert(2, pltpu.VMEM((NCH, CH), jnp.int32))
    else:
        scratch.insert(2, pltpu.VMEM((CH,), jnp.int32))
        scratch.insert(3, pltpu.VMEM((CH,), jnp.int32))
    if with_deg:
        out_type.append(jax.ShapeDtypeStruct((NC, NPAD), _f32))
        out_type.append(jax.ShapeDtypeStruct((NW, NPAD), _f32))
        scratch += [
            pltpu.VMEM((NPAD,), _f32),
            pltpu.VMEM((RPS,), _f32),
            pltpu.VMEM((RPS,), _f32),
        ]

    @functools.partial(
        pl.kernel, out_type=tuple(out_type), mesh=mesh,
        scratch_types=scratch,
        compiler_params=pltpu.CompilerParams(needs_layout_passes=False))
    def agg(h_hbm, src_hbm, dst_hbm, *rest):
        it = iter(rest)
        out_hbm = next(it)
        if with_deg:
            deg_hbm = next(it)
            stage_hbm = next(it)
        sidx0 = next(it)
        sidx1 = next(it)
        if preload_dst:
            didx2 = next(it)
        else:
            didx0 = next(it)
            didx1 = next(it)
        rows0 = next(it)
        rows1 = next(it)
        acc = next(it)
        gsem0 = next(it)
        gsem1 = next(it)
        ssem0 = next(it)
        ssem1 = next(it)
        isem = next(it)
        if with_deg:
            hist = next(it)
            drow = next(it)
            degv = next(it)

        c = lax.axis_index("c")
        s = lax.axis_index("s")
        wid = s * NC + c
        base = wid * EWP

        if preload_dst:
            pltpu.async_copy(dst_hbm.at[wid], didx2, isem)

        def zrow(j, carry):
            idx = j * 16
            rows0[idx // D, pl.ds(idx % D, 16)] = jnp.zeros((16,), _f32)
            return carry
        lax.fori_loop(0, ZR * D // 16, zrow, 0)
        for k in range(RPS // ZR):
            pltpu.sync_copy(rows0.at[pl.ds(0, ZR)],
                            acc.at[pl.ds(s * RPS + k * ZR, ZR)])

        if with_deg:
            def zhist(j, carry):
                hist[pl.ds(j * 16, 16)] = jnp.zeros((16,), _f32)
                return carry
            lax.fori_loop(0, NPAD // 16, zhist, 0)
            ones = jnp.ones((16,), _f32)

        if preload_dst:
            pltpu.make_async_copy(dst_hbm.at[wid], didx2, isem).wait()
        plsc.subcore_barrier()

        def count(idx_buf):
            if with_deg:
                for k in range(CH // 16):
                    dv = idx_buf[pl.ds(k * 16, 16)]
                    plsc.addupdate_scatter(hist, [dv], ones)

        pltpu.sync_copy(src_hbm.at[pl.ds(base, CH)], sidx0)
        pltpu.async_copy(h_hbm.at[sidx0], rows0, gsem0)
        pltpu.sync_copy(src_hbm.at[pl.ds(base + CH, CH)], sidx1)
        pltpu.async_copy(h_hbm.at[sidx1], rows1, gsem1)

        def body(i, carry):
            j0 = 2 * i
            j1 = j0 + 1
            if preload_dst:
                d0 = didx2.at[j0]
            else:
                pltpu.sync_copy(dst_hbm.at[pl.ds(base + j0 * CH, CH)], didx0)
                d0 = didx0
                count(didx0)
            pltpu.make_async_copy(h_hbm.at[sidx0], rows0, gsem0).wait()
            pltpu.async_copy(rows0, acc.at[d0], ssem0, add=True)
            if preload_dst:
                d1 = didx2.at[j1]
            else:
                pltpu.sync_copy(dst_hbm.at[pl.ds(base + j1 * CH, CH)], didx1)
                d1 = didx1
                count(didx1)
            pltpu.make_async_copy(h_hbm.at[sidx1], rows1, gsem1).wait()
            pltpu.async_copy(rows1, acc.at[d1], ssem1, add=True)
            @pl.when(j0 + 2 < NCH)
            def _():
                pltpu.make_async_copy(rows0, acc.at[d0], ssem0).wait()
                pltpu.sync_copy(
                    src_hbm.at[pl.ds(base + (j0 + 2) * CH, CH)], sidx0)
                pltpu.async_copy(h_hbm.at[sidx0], rows0, gsem0)

            @pl.when(j1 + 2 < NCH)
            def _():
                pltpu.make_async_copy(rows1, acc.at[d1], ssem1).wait()
                pltpu.sync_copy(
                    src_hbm.at[pl.ds(base + (j1 + 2) * CH, CH)], sidx1)
                pltpu.async_copy(h_hbm.at[sidx1], rows1, gsem1)
            return carry

        lax.fori_loop(0, NCH // 2, body, 0)
        if preload_dst:
            dd = didx2.at[0]
        else:
            dd = didx0
        pltpu.make_async_copy(rows0, acc.at[dd], ssem0).wait()
        pltpu.make_async_copy(rows1, acc.at[dd], ssem1).wait()

        if with_deg:
            pltpu.sync_copy(hist, stage_hbm.at[wid])
            plsc.subcore_barrier()

            def zdeg(j, carry):
                degv[pl.ds(j * 16, 16)] = jnp.zeros((16,), _f32)
                return carry
            lax.fori_loop(0, RPS // 16, zdeg, 0)
            for r in range(NS):
                pltpu.sync_copy(
                    stage_hbm.at[r * NC + c, pl.ds(s * RPS, RPS)], drow)

                def dbody(j, carry):
                    col = j * 16
                    degv[pl.ds(col, 16)] += drow[pl.ds(col, 16)]
                    return carry
                lax.fori_loop(0, RPS // 16, dbody, 0)
            pltpu.sync_copy(degv, deg_hbm.at[c, pl.ds(s * RPS, RPS)])

        plsc.subcore_barrier()
        pltpu.sync_copy(
            acc.at[pl.ds(s * RPS, RPS)],
            out_hbm.at[pl.ds(c * NPAD + s * RPS, RPS)])

    return agg


_sc_agg_deg = _make_sc_agg(True, False)
_sc_agg = _make_sc_agg(False, False)


def _pad_idx(v, fill):
    per = v.reshape(NW, EW)
    pad = jnp.broadcast_to(fill.reshape(1, EWP - EW), (NW, EWP - EW))
    return jnp.concatenate([per, pad], axis=1).reshape(NW * EWP)

BS = 400
NBLK = N // BS


def _conv_body(h_ref, part_ref, deg_ref, wr_ref, wn_ref, b_ref,
               gamma_ref, beta_ref, out_ref, hpre_s, sum_s, ssq_s):
    p = pl.program_id(0)
    i = pl.program_id(1)

    @pl.when(p == 0)
    def _():
        agg = part_ref[0] + part_ref[1]
        mean = agg / jnp.maximum(deg_ref[...], 1.0)
        hp = (jnp.dot(h_ref[...], wr_ref[...], preferred_element_type=_f32)
              + jnp.dot(mean, wn_ref[...], preferred_element_type=_f32)
              + b_ref[...])
        hpre_s[pl.ds(i * BS, BS), :] = hp

        @pl.when(i == 0)
        def _():
            sum_s[...] = jnp.zeros_like(sum_s)
            ssq_s[...] = jnp.zeros_like(ssq_s)

        sum_s[...] += jnp.sum(hp, axis=0, keepdims=True)
        ssq_s[...] += jnp.sum(hp * hp, axis=0, keepdims=True)

    @pl.when(p == 1)
    def _():
        mu = sum_s[...] / N
        var = ssq_s[...] / N - mu * mu
        rstd = lax.rsqrt(var + 1e-5)
        hp = hpre_s[pl.ds(i * BS, BS), :]
        hb = (hp - mu) * (rstd * gamma_ref[...]) + beta_ref[...]
        out_ref[...] = jnp.maximum(hb, 0.0)


def _conv(h, part, deg, wr, wn, b, gamma, beta):
    def pin(f):
        return lambda p, i: f(jnp.where(p == 0, i, 0))
    return pl.pallas_call(
        _conv_body,
        grid=(2, NBLK),
        in_specs=[
            pl.BlockSpec((BS, D), pin(lambda i: (i, 0))),
            pl.BlockSpec((NC, BS, D), pin(lambda i: (0, i, 0))),
            pl.BlockSpec((BS, 1), pin(lambda i: (i, 0))),
            pl.BlockSpec((D, D), lambda p, i: (0, 0)),
            pl.BlockSpec((D, D), lambda p, i: (0, 0)),
            pl.BlockSpec((1, D), lambda p, i: (0, 0)),
            pl.BlockSpec((1, D), lambda p, i: (0, 0)),
            pl.BlockSpec((1, D), lambda p, i: (0, 0)),
        ],
        out_specs=pl.BlockSpec((BS, D), lambda p, i: (i, 0)),
        out_shape=jax.ShapeDtypeStruct((N, D), _f32),
        scratch_shapes=[
            pltpu.VMEM((N, D), _f32),
            pltpu.VMEM((1, D), _f32),
            pltpu.VMEM((1, D), _f32),
        ],
    )(h, part, deg, wr, wn, b, gamma, beta)


def _head_body(xn_ref, w1_ref, b1_ref, w2_ref, b2_ref, w3_ref, b3_ref,
               gsw1_ref, gsb1_ref, gsw2_ref, gsb2_ref,
               ghw1_ref, ghb1_ref, ghw2_ref, ghb2_ref, ghw3_ref, ghb3_ref,
               gout_ref, nout_ref):
    xn = xn_ref[...]
    h1 = lax.dot_general(xn, w1_ref[...], (((2,), (1,)), ((1,), (0,))),
                         preferred_element_type=_f32)
    h1 = jnp.maximum(h1 + b1_ref[...][:, None, :], 0.0)
    h2 = lax.dot_general(h1, w2_ref[...], (((2,), (1,)), ((0,), (0,))),
                         preferred_element_type=_f32)
    h2 = jnp.maximum(h2 + b2_ref[...][:, None, :], 0.0)
    w3 = w3_ref[...][:, :, 0]
    nout_ref[...] = jnp.sum(h2 * w3[:, None, :], axis=2) + b3_ref[...]

    g = jnp.maximum(jnp.sum(xn, axis=1) / NN, 0.0)
    g = jnp.dot(g, gsw1_ref[...], preferred_element_type=_f32) + gsb1_ref[...]
    g = jnp.dot(g, gsw2_ref[...], preferred_element_type=_f32) + gsb2_ref[...]
    g = jnp.maximum(g, 0.0)
    g = jnp.maximum(
        jnp.dot(g, ghw1_ref[...], preferred_element_type=_f32) + ghb1_ref[...], 0.0)
    g = jnp.maximum(
        jnp.dot(g, ghw2_ref[...], preferred_element_type=_f32) + ghb2_ref[...], 0.0)
    gout_ref[...] = (jnp.dot(g, ghw3_ref[...], preferred_element_type=_f32)
                     + ghb3_ref[...])


def _head(xn, nh_W1, nh_b1, nh_W2, nh_b2, nh_W3, nh_b3,
          gs_W1, gs_b1, gs_W2, gs_b2,
          gh_W1, gh_b1, gh_W2, gh_b2, gh_W3, gh_b3):
    return pl.pallas_call(
        _head_body,
        out_shape=[
            jax.ShapeDtypeStruct((B, 1), _f32),
            jax.ShapeDtypeStruct((NN, B), _f32),
        ],
    )(xn, nh_W1, nh_b1, nh_W2, nh_b2, nh_W3, nh_b3,
      gs_W1, gs_b1, gs_W2, gs_b2, gh_W1, gh_b1, gh_W2, gh_b2, gh_W3, gh_b3)


def kernel(x, conv_Wr, conv_Wn, conv_b, bn_gamma, bn_beta,
           gs_W1, gs_b1, gs_W2, gs_b2,
           gh_W1, gh_b1, gh_W2, gh_b2, gh_W3, gh_b3,
           nh_W1, nh_b1, nh_W2, nh_b2, nh_W3, nh_b3,
           edge_index, batch):
    src = edge_index[0]
    dst = edge_index[1]

    padn = EWP - EW
    sfill = (jnp.arange(padn, dtype=jnp.int32) * 997) % N
    dfill = N + (jnp.arange(padn, dtype=jnp.int32) % (NPAD - N))
    srcf = _pad_idx(src, sfill)
    dstf = _pad_idx(dst, dfill)
    dst2 = dstf.reshape(NW, NCH, CH)

    part0_flat, degp, _ = _sc_agg_deg(x, srcf, dstf)
    part0 = part0_flat.reshape(NC, NPAD, D)
    deg = (degp[0] + degp[1])[:N].reshape(N, 1)
    h1 = _conv(x, part0, deg, conv_Wr[0], conv_Wn[0], conv_b[0].reshape(1, D),
               bn_gamma[0:1], bn_beta[0:1])

    part1 = _sc_agg(h1, srcf, dstf)[0].reshape(NC, NPAD, D)
    h2 = _conv(h1, part1, deg, conv_Wr[1], conv_Wn[1], conv_b[1].reshape(1, D),
               bn_gamma[1:2], bn_beta[1:2])

    g_out, n_outT = _head(
        h2.reshape(B, NN, D), nh_W1, nh_b1, nh_W2, nh_b2, nh_W3, nh_b3,
        gs_W1, gs_b1.reshape(1, DS), gs_W2, gs_b2.reshape(1, DS),
        gh_W1, gh_b1.reshape(1, DH), gh_W2, gh_b2.reshape(1, DH),
        gh_W3, gh_b3.reshape(1, 1))
    return jnp.concatenate([g_out, n_outT.T], axis=1)


# speedup vs baseline: 1.0140x; 1.0140x over previous
import functools

import jax
import jax.numpy as jnp
from jax import lax
from jax.experimental import pallas as pl
from jax.experimental.pallas import tpu as pltpu
from jax.experimental.pallas import tpu_sc as plsc

N = 10000
E = 320000
D = 128
B = 100
NN = 100
DS = 64
DH = 64

NC, NS = 2, 16
NW = NC * NS
EW = E // NW

NPAD = 10240
RPS = NPAD // NS
ZR = 64

NCH = 80
CH = 128
EWP = NCH * CH

_f32 = jnp.float32


def _make_sc_agg(with_deg, preload_dst):
    mesh = plsc.VectorSubcoreMesh(core_axis_name="c", subcore_axis_name="s")

    out_type = [jax.ShapeDtypeStruct((NC * NPAD, D), _f32)]
    scratch = [
        pltpu.VMEM((CH,), jnp.int32),
        pltpu.VMEM((CH,), jnp.int32),
        pltpu.VMEM((CH, D), _f32),
        pltpu.VMEM((CH, D), _f32),
        pltpu.VMEM_SHARED((NPAD, D), _f32),
        pltpu.SemaphoreType.DMA,
        pltpu.SemaphoreType.DMA,
        pltpu.SemaphoreType.DMA,
        pltpu.SemaphoreType.DMA,
        pltpu.SemaphoreType.DMA,
    ]
    if preload_dst:
        scratch.insert(2, pltpu.VMEM((NCH, CH), jnp.int32))
    else:
        scratch.insert(2, pltpu.VMEM((CH,), jnp.int32))
        scratch.insert(3, pltpu.VMEM((CH,), jnp.int32))
    if with_deg:
        out_type.append(jax.ShapeDtypeStruct((NC, NPAD), _f32))
        out_type.append(jax.ShapeDtypeStruct((NW, NPAD), _f32))
        scratch += [
            pltpu.VMEM((NPAD,), _f32),
            pltpu.VMEM((RPS,), _f32),
            pltpu.VMEM((RPS,), _f32),
        ]

    @functools.partial(
        pl.kernel, out_type=tuple(out_type), mesh=mesh,
        scratch_types=scratch,
        compiler_params=pltpu.CompilerParams(needs_layout_passes=False))
    def agg(h_hbm, src_hbm, dst_hbm, *rest):
        it = iter(rest)
        out_hbm = next(it)
        if with_deg:
            deg_hbm = next(it)
            stage_hbm = next(it)
        sidx0 = next(it)
        sidx1 = next(it)
        if preload_dst:
            didx2 = next(it)
        else:
            didx0 = next(it)
            didx1 = next(it)
        rows0 = next(it)
        rows1 = next(it)
        acc = next(it)
        gsem0 = next(it)
        gsem1 = next(it)
        ssem0 = next(it)
        ssem1 = next(it)
        isem = next(it)
        if with_deg:
            hist = next(it)
            drow = next(it)
            degv = next(it)

        c = lax.axis_index("c")
        s = lax.axis_index("s")
        wid = s * NC + c
        base = wid * EWP

        if preload_dst:
            pltpu.async_copy(dst_hbm.at[wid], didx2, isem)

        def zrow(j, carry):
            idx = j * 16
            rows0[idx // D, pl.ds(idx % D, 16)] = jnp.zeros((16,), _f32)
            return carry
        lax.fori_loop(0, ZR * D // 16, zrow, 0)
        for k in range(RPS // ZR):
            pltpu.sync_copy(rows0.at[pl.ds(0, ZR)],
                            acc.at[pl.ds(s * RPS + k * ZR, ZR)])

        if with_deg:
            def zhist(j, carry):
                hist[pl.ds(j * 16, 16)] = jnp.zeros((16,), _f32)
                return carry
            lax.fori_loop(0, NPAD // 16, zhist, 0)
            ones = jnp.ones((16,), _f32)

        if preload_dst:
            pltpu.make_async_copy(dst_hbm.at[wid], didx2, isem).wait()
        plsc.subcore_barrier()

        def count(idx_buf):
            if with_deg:
                for k in range(CH // 16):
                    dv = idx_buf[pl.ds(k * 16, 16)]
                    plsc.addupdate_scatter(hist, [dv], ones)

        pltpu.sync_copy(src_hbm.at[pl.ds(base, CH)], sidx0)
        pltpu.async_copy(h_hbm.at[sidx0], rows0, gsem0)
        pltpu.sync_copy(src_hbm.at[pl.ds(base + CH, CH)], sidx1)
        pltpu.async_copy(h_hbm.at[sidx1], rows1, gsem1)

        def body(i, carry):
            j0 = 2 * i
            j1 = j0 + 1
            if preload_dst:
                d0 = didx2.at[j0]
            else:
                pltpu.sync_copy(dst_hbm.at[pl.ds(base + j0 * CH, CH)], didx0)
                d0 = didx0
                count(didx0)
            pltpu.make_async_copy(h_hbm.at[sidx0], rows0, gsem0).wait()
            pltpu.async_copy(rows0, acc.at[d0], ssem0, add=True)
            if preload_dst:
                d1 = didx2.at[j1]
            else:
                pltpu.sync_copy(dst_hbm.at[pl.ds(base + j1 * CH, CH)], didx1)
                d1 = didx1
                count(didx1)
            pltpu.make_async_copy(h_hbm.at[sidx1], rows1, gsem1).wait()
            pltpu.async_copy(rows1, acc.at[d1], ssem1, add=True)
            @pl.when(j0 + 2 < NCH)
            def _():
                pltpu.make_async_copy(rows0, acc.at[d0], ssem0).wait()
                pltpu.sync_copy(
                    src_hbm.at[pl.ds(base + (j0 + 2) * CH, CH)], sidx0)
                pltpu.async_copy(h_hbm.at[sidx0], rows0, gsem0)

            @pl.when(j1 + 2 < NCH)
            def _():
                pltpu.make_async_copy(rows1, acc.at[d1], ssem1).wait()
                pltpu.sync_copy(
                    src_hbm.at[pl.ds(base + (j1 + 2) * CH, CH)], sidx1)
                pltpu.async_copy(h_hbm.at[sidx1], rows1, gsem1)
            return carry

        lax.fori_loop(0, NCH // 2, body, 0)
        if preload_dst:
            dd = didx2.at[0]
        else:
            dd = didx0
        pltpu.make_async_copy(rows0, acc.at[dd], ssem0).wait()
        pltpu.make_async_copy(rows1, acc.at[dd], ssem1).wait()

        if with_deg:
            pltpu.sync_copy(hist, stage_hbm.at[wid])
            plsc.subcore_barrier()
            pltpu.async_copy(
                acc.at[pl.ds(s * RPS, RPS)],
                out_hbm.at[pl.ds(c * NPAD + s * RPS, RPS)], isem)

            def zdeg(j, carry):
                degv[pl.ds(j * 16, 16)] = jnp.zeros((16,), _f32)
                return carry
            lax.fori_loop(0, RPS // 16, zdeg, 0)
            for r in range(NS):
                pltpu.sync_copy(
                    stage_hbm.at[r * NC + c, pl.ds(s * RPS, RPS)], drow)

                def dbody(j, carry):
                    col = j * 16
                    degv[pl.ds(col, 16)] += drow[pl.ds(col, 16)]
                    return carry
                lax.fori_loop(0, RPS // 16, dbody, 0)
            pltpu.sync_copy(degv, deg_hbm.at[c, pl.ds(s * RPS, RPS)])
            pltpu.make_async_copy(
                acc.at[pl.ds(s * RPS, RPS)],
                out_hbm.at[pl.ds(c * NPAD + s * RPS, RPS)], isem).wait()
        else:
            plsc.subcore_barrier()
            pltpu.sync_copy(
                acc.at[pl.ds(s * RPS, RPS)],
                out_hbm.at[pl.ds(c * NPAD + s * RPS, RPS)])

    return agg


_sc_agg_deg = _make_sc_agg(True, False)
_sc_agg = _make_sc_agg(False, False)


def _pad_idx(v, fill):
    per = v.reshape(NW, EW)
    pad = jnp.broadcast_to(fill.reshape(1, EWP - EW), (NW, EWP - EW))
    return jnp.concatenate([per, pad], axis=1).reshape(NW * EWP)

BS = 400
NBLK = N // BS


def _conv_body(h_ref, part_ref, deg_ref, wr_ref, wn_ref, b_ref,
               gamma_ref, beta_ref, out_ref, hpre_s, sum_s, ssq_s):
    p = pl.program_id(0)
    i = pl.program_id(1)

    @pl.when(p == 0)
    def _():
        agg = part_ref[0] + part_ref[1]
        mean = agg / jnp.maximum(deg_ref[...], 1.0)
        hp = (jnp.dot(h_ref[...], wr_ref[...], preferred_element_type=_f32)
              + jnp.dot(mean, wn_ref[...], preferred_element_type=_f32)
              + b_ref[...])
        hpre_s[pl.ds(i * BS, BS), :] = hp

        @pl.when(i == 0)
        def _():
            sum_s[...] = jnp.zeros_like(sum_s)
            ssq_s[...] = jnp.zeros_like(ssq_s)

        sum_s[...] += jnp.sum(hp, axis=0, keepdims=True)
        ssq_s[...] += jnp.sum(hp * hp, axis=0, keepdims=True)

    @pl.when(p == 1)
    def _():
        mu = sum_s[...] / N
        var = ssq_s[...] / N - mu * mu
        rstd = lax.rsqrt(var + 1e-5)
        hp = hpre_s[pl.ds(i * BS, BS), :]
        hb = (hp - mu) * (rstd * gamma_ref[...]) + beta_ref[...]
        out_ref[...] = jnp.maximum(hb, 0.0)


def _conv(h, part, deg, wr, wn, b, gamma, beta):
    def pin(f):
        return lambda p, i: f(jnp.where(p == 0, i, 0))
    return pl.pallas_call(
        _conv_body,
        grid=(2, NBLK),
        in_specs=[
            pl.BlockSpec((BS, D), pin(lambda i: (i, 0))),
            pl.BlockSpec((NC, BS, D), pin(lambda i: (0, i, 0))),
            pl.BlockSpec((BS, 1), pin(lambda i: (i, 0))),
            pl.BlockSpec((D, D), lambda p, i: (0, 0)),
            pl.BlockSpec((D, D), lambda p, i: (0, 0)),
            pl.BlockSpec((1, D), lambda p, i: (0, 0)),
            pl.BlockSpec((1, D), lambda p, i: (0, 0)),
            pl.BlockSpec((1, D), lambda p, i: (0, 0)),
        ],
        out_specs=pl.BlockSpec((BS, D), lambda p, i: (i, 0)),
        out_shape=jax.ShapeDtypeStruct((N, D), _f32),
        scratch_shapes=[
            pltpu.VMEM((N, D), _f32),
            pltpu.VMEM((1, D), _f32),
            pltpu.VMEM((1, D), _f32),
        ],
    )(h, part, deg, wr, wn, b, gamma, beta)


def _head_body(xn_ref, w1_ref, b1_ref, w2_ref, b2_ref, w3_ref, b3_ref,
               gsw1_ref, gsb1_ref, gsw2_ref, gsb2_ref,
               ghw1_ref, ghb1_ref, ghw2_ref, ghb2_ref, ghw3_ref, ghb3_ref,
               gout_ref, nout_ref):
    xn = xn_ref[...]
    h1 = lax.dot_general(xn, w1_ref[...], (((2,), (1,)), ((1,), (0,))),
                         preferred_element_type=_f32)
    h1 = jnp.maximum(h1 + b1_ref[...][:, None, :], 0.0)
    h2 = lax.dot_general(h1, w2_ref[...], (((2,), (1,)), ((0,), (0,))),
                         preferred_element_type=_f32)
    h2 = jnp.maximum(h2 + b2_ref[...][:, None, :], 0.0)
    w3 = w3_ref[...][:, :, 0]
    nout_ref[...] = jnp.sum(h2 * w3[:, None, :], axis=2) + b3_ref[...]

    g = jnp.maximum(jnp.sum(xn, axis=1) / NN, 0.0)
    g = jnp.dot(g, gsw1_ref[...], preferred_element_type=_f32) + gsb1_ref[...]
    g = jnp.dot(g, gsw2_ref[...], preferred_element_type=_f32) + gsb2_ref[...]
    g = jnp.maximum(g, 0.0)
    g = jnp.maximum(
        jnp.dot(g, ghw1_ref[...], preferred_element_type=_f32) + ghb1_ref[...], 0.0)
    g = jnp.maximum(
        jnp.dot(g, ghw2_ref[...], preferred_element_type=_f32) + ghb2_ref[...], 0.0)
    gout_ref[...] = (jnp.dot(g, ghw3_ref[...], preferred_element_type=_f32)
                     + ghb3_ref[...])


def _head(xn, nh_W1, nh_b1, nh_W2, nh_b2, nh_W3, nh_b3,
          gs_W1, gs_b1, gs_W2, gs_b2,
          gh_W1, gh_b1, gh_W2, gh_b2, gh_W3, gh_b3):
    return pl.pallas_call(
        _head_body,
        out_shape=[
            jax.ShapeDtypeStruct((B, 1), _f32),
            jax.ShapeDtypeStruct((NN, B), _f32),
        ],
    )(xn, nh_W1, nh_b1, nh_W2, nh_b2, nh_W3, nh_b3,
      gs_W1, gs_b1, gs_W2, gs_b2, gh_W1, gh_b1, gh_W2, gh_b2, gh_W3, gh_b3)


def kernel(x, conv_Wr, conv_Wn, conv_b, bn_gamma, bn_beta,
           gs_W1, gs_b1, gs_W2, gs_b2,
           gh_W1, gh_b1, gh_W2, gh_b2, gh_W3, gh_b3,
           nh_W1, nh_b1, nh_W2, nh_b2, nh_W3, nh_b3,
           edge_index, batch):
    src = edge_index[0]
    dst = edge_index[1]

    padn = EWP - EW
    sfill = (jnp.arange(padn, dtype=jnp.int32) * 997) % N
    dfill = N + (jnp.arange(padn, dtype=jnp.int32) % (NPAD - N))
    srcf = _pad_idx(src, sfill)
    dstf = _pad_idx(dst, dfill)
    dst2 = dstf.reshape(NW, NCH, CH)

    part0_flat, degp, _ = _sc_agg_deg(x, srcf, dstf)
    part0 = part0_flat.reshape(NC, NPAD, D)
    deg = (degp[0] + degp[1])[:N].reshape(N, 1)
    h1 = _conv(x, part0, deg, conv_Wr[0], conv_Wn[0], conv_b[0].reshape(1, D),
               bn_gamma[0:1], bn_beta[0:1])

    part1 = _sc_agg(h1, srcf, dstf)[0].reshape(NC, NPAD, D)
    h2 = _conv(h1, part1, deg, conv_Wr[1], conv_Wn[1], conv_b[1].reshape(1, D),
               bn_gamma[1:2], bn_beta[1:2])

    g_out, n_outT = _head(
        h2.reshape(B, NN, D), nh_W1, nh_b1, nh_W2, nh_b2, nh_W3, nh_b3,
        gs_W1, gs_b1.reshape(1, DS), gs_W2, gs_b2.reshape(1, DS),
        gh_W1, gh_b1.reshape(1, DH), gh_W2, gh_b2.reshape(1, DH),
        gh_W3, gh_b3.reshape(1, 1))
    return jnp.concatenate([g_out, n_outT.T], axis=1)


# speedup vs baseline: 1.0798x; 1.0648x over previous
import functools

import jax
import jax.numpy as jnp
from jax import lax
from jax.experimental import pallas as pl
from jax.experimental.pallas import tpu as pltpu
from jax.experimental.pallas import tpu_sc as plsc

N = 10000
E = 320000
D = 128
B = 100
NN = 100
DS = 64
DH = 64

NC, NS = 2, 16
NW = NC * NS
EW = E // NW

NPAD = 10240
RPS = NPAD // NS
ZR = 64

NCH = 80
CH = 128
EWP = NCH * CH

_f32 = jnp.float32


def _make_sc_agg(with_deg, preload_dst):
    mesh = plsc.VectorSubcoreMesh(core_axis_name="c", subcore_axis_name="s")

    out_type = [jax.ShapeDtypeStruct((NC * NPAD, D), _f32)]
    scratch = [
        pltpu.VMEM((CH,), jnp.int32),
        pltpu.VMEM((CH,), jnp.int32),
        pltpu.VMEM((CH, D), _f32),
        pltpu.VMEM((CH, D), _f32),
        pltpu.VMEM_SHARED((NPAD, D), _f32),
        pltpu.SemaphoreType.DMA,
        pltpu.SemaphoreType.DMA,
        pltpu.SemaphoreType.DMA,
        pltpu.SemaphoreType.DMA,
        pltpu.SemaphoreType.DMA,
    ]
    if preload_dst:
        scratch.insert(2, pltpu.VMEM((NCH, CH), jnp.int32))
    else:
        scratch.insert(2, pltpu.VMEM((CH,), jnp.int32))
        scratch.insert(3, pltpu.VMEM((CH,), jnp.int32))
    if with_deg:
        out_type.append(jax.ShapeDtypeStruct((NC, NPAD), _f32))
        out_type.append(jax.ShapeDtypeStruct((NW, NPAD), _f32))
        scratch += [
            pltpu.VMEM((NPAD,), _f32),
            pltpu.VMEM((RPS,), _f32),
            pltpu.VMEM((RPS,), _f32),
        ]

    @functools.partial(
        pl.kernel, out_type=tuple(out_type), mesh=mesh,
        scratch_types=scratch,
        compiler_params=pltpu.CompilerParams(needs_layout_passes=False))
    def agg(h_hbm, src_hbm, dst_hbm, *rest):
        it = iter(rest)
        out_hbm = next(it)
        if with_deg:
            deg_hbm = next(it)
            stage_hbm = next(it)
        sidx0 = next(it)
        sidx1 = next(it)
        if preload_dst:
            didx2 = next(it)
        else:
            didx0 = next(it)
            didx1 = next(it)
        rows0 = next(it)
        rows1 = next(it)
        acc = next(it)
        gsem0 = next(it)
        gsem1 = next(it)
        ssem0 = next(it)
        ssem1 = next(it)
        isem = next(it)
        if with_deg:
            hist = next(it)
            drow = next(it)
            degv = next(it)

        c = lax.axis_index("c")
        s = lax.axis_index("s")
        wid = s * NC + c
        base = wid * EWP

        if preload_dst:
            pltpu.async_copy(dst_hbm.at[wid], didx2, isem)

        def zrow(j, carry):
            idx = j * 16
            rows0[idx // D, pl.ds(idx % D, 16)] = jnp.zeros((16,), _f32)
            return carry
        lax.fori_loop(0, ZR * D // 16, zrow, 0)
        for k in range(RPS // ZR):
            pltpu.sync_copy(rows0.at[pl.ds(0, ZR)],
                            acc.at[pl.ds(s * RPS + k * ZR, ZR)])

        if with_deg:
            def zhist(j, carry):
                hist[pl.ds(j * 16, 16)] = jnp.zeros((16,), _f32)
                return carry
            lax.fori_loop(0, NPAD // 16, zhist, 0)
            ones = jnp.ones((16,), _f32)

        if preload_dst:
            pltpu.make_async_copy(dst_hbm.at[wid], didx2, isem).wait()
        plsc.subcore_barrier()

        def count(idx_buf):
            if with_deg:
                for k in range(CH // 16):
                    dv = idx_buf[pl.ds(k * 16, 16)]
                    plsc.addupdate_scatter(hist, [dv], ones)

        pltpu.sync_copy(src_hbm.at[pl.ds(base, CH)], sidx0)
        pltpu.async_copy(h_hbm.at[sidx0], rows0, gsem0)
        pltpu.sync_copy(src_hbm.at[pl.ds(base + CH, CH)], sidx1)
        pltpu.async_copy(h_hbm.at[sidx1], rows1, gsem1)

        def body(i, carry):
            j0 = 2 * i
            j1 = j0 + 1
            if preload_dst:
                d0 = didx2.at[j0]
            else:
                pltpu.sync_copy(dst_hbm.at[pl.ds(base + j0 * CH, CH)], didx0)
                d0 = didx0
                count(didx0)
            pltpu.make_async_copy(h_hbm.at[sidx0], rows0, gsem0).wait()
            pltpu.async_copy(rows0, acc.at[d0], ssem0, add=True)
            if preload_dst:
                d1 = didx2.at[j1]
            else:
                pltpu.sync_copy(dst_hbm.at[pl.ds(base + j1 * CH, CH)], didx1)
                d1 = didx1
                count(didx1)
            pltpu.make_async_copy(h_hbm.at[sidx1], rows1, gsem1).wait()
            pltpu.async_copy(rows1, acc.at[d1], ssem1, add=True)
            @pl.when(j0 + 2 < NCH)
            def _():
                pltpu.make_async_copy(rows0, acc.at[d0], ssem0).wait()
                pltpu.sync_copy(
                    src_hbm.at[pl.ds(base + (j0 + 2) * CH, CH)], sidx0)
                pltpu.async_copy(h_hbm.at[sidx0], rows0, gsem0)

            @pl.when(j1 + 2 < NCH)
            def _():
                pltpu.make_async_copy(rows1, acc.at[d1], ssem1).wait()
                pltpu.sync_copy(
                    src_hbm.at[pl.ds(base + (j1 + 2) * CH, CH)], sidx1)
                pltpu.async_copy(h_hbm.at[sidx1], rows1, gsem1)
            return carry

        lax.fori_loop(0, NCH // 2, body, 0)
        if preload_dst:
            dd = didx2.at[0]
        else:
            dd = didx0
        pltpu.make_async_copy(rows0, acc.at[dd], ssem0).wait()
        pltpu.make_async_copy(rows1, acc.at[dd], ssem1).wait()

        if with_deg:
            pltpu.sync_copy(hist, stage_hbm.at[wid])
            plsc.subcore_barrier()
            pltpu.async_copy(
                acc.at[pl.ds(s * RPS, RPS)],
                out_hbm.at[pl.ds(c * NPAD + s * RPS, RPS)], isem)

            def zdeg(j, carry):
                degv[pl.ds(j * 16, 16)] = jnp.zeros((16,), _f32)
                return carry
            lax.fori_loop(0, RPS // 16, zdeg, 0)
            for r in range(NS):
                pltpu.sync_copy(
                    stage_hbm.at[r * NC + c, pl.ds(s * RPS, RPS)], drow)

                def dbody(j, carry):
                    col = j * 16
                    degv[pl.ds(col, 16)] += drow[pl.ds(col, 16)]
                    return carry
                lax.fori_loop(0, RPS // 16, dbody, 0)
            pltpu.sync_copy(degv, deg_hbm.at[c, pl.ds(s * RPS, RPS)])
            pltpu.make_async_copy(
                acc.at[pl.ds(s * RPS, RPS)],
                out_hbm.at[pl.ds(c * NPAD + s * RPS, RPS)], isem).wait()
        else:
            plsc.subcore_barrier()
            pltpu.sync_copy(
                acc.at[pl.ds(s * RPS, RPS)],
                out_hbm.at[pl.ds(c * NPAD + s * RPS, RPS)])

    return agg


_sc_agg_deg = _make_sc_agg(True, False)
_sc_agg = _make_sc_agg(False, False)


def _pad_idx(v, fill):
    per = v.reshape(NW, EW)
    pad = jnp.broadcast_to(fill.reshape(1, EWP - EW), (NW, EWP - EW))
    return jnp.concatenate([per, pad], axis=1).reshape(NW * EWP)

BS = 1000
NBLK = N // BS


def _conv_body(h_ref, part_ref, deg_ref, wr_ref, wn_ref, b_ref,
               gamma_ref, beta_ref, out_ref, hpre_s, sum_s, ssq_s):
    p = pl.program_id(0)
    i = pl.program_id(1)

    @pl.when(p == 0)
    def _():
        agg = part_ref[0] + part_ref[1]
        mean = agg / jnp.maximum(deg_ref[...], 1.0)
        hp = (jnp.dot(h_ref[...], wr_ref[...], preferred_element_type=_f32)
              + jnp.dot(mean, wn_ref[...], preferred_element_type=_f32)
              + b_ref[...])
        hpre_s[pl.ds(i * BS, BS), :] = hp

        @pl.when(i == 0)
        def _():
            sum_s[...] = jnp.zeros_like(sum_s)
            ssq_s[...] = jnp.zeros_like(ssq_s)

        sum_s[...] += jnp.sum(hp, axis=0, keepdims=True)
        ssq_s[...] += jnp.sum(hp * hp, axis=0, keepdims=True)

    @pl.when(p == 1)
    def _():
        mu = sum_s[...] / N
        var = ssq_s[...] / N - mu * mu
        rstd = lax.rsqrt(var + 1e-5)
        hp = hpre_s[pl.ds(i * BS, BS), :]
        hb = (hp - mu) * (rstd * gamma_ref[...]) + beta_ref[...]
        out_ref[...] = jnp.maximum(hb, 0.0)


def _conv(h, part, deg, wr, wn, b, gamma, beta):
    def pin(f):
        return lambda p, i: f(jnp.where(p == 0, i, 0))
    return pl.pallas_call(
        _conv_body,
        grid=(2, NBLK),
        in_specs=[
            pl.BlockSpec((BS, D), pin(lambda i: (i, 0))),
            pl.BlockSpec((NC, BS, D), pin(lambda i: (0, i, 0))),
            pl.BlockSpec((BS, 1), pin(lambda i: (i, 0))),
            pl.BlockSpec((D, D), lambda p, i: (0, 0)),
            pl.BlockSpec((D, D), lambda p, i: (0, 0)),
            pl.BlockSpec((1, D), lambda p, i: (0, 0)),
            pl.BlockSpec((1, D), lambda p, i: (0, 0)),
            pl.BlockSpec((1, D), lambda p, i: (0, 0)),
        ],
        out_specs=pl.BlockSpec((BS, D), lambda p, i: (i, 0)),
        out_shape=jax.ShapeDtypeStruct((N, D), _f32),
        scratch_shapes=[
            pltpu.VMEM((N, D), _f32),
            pltpu.VMEM((1, D), _f32),
            pltpu.VMEM((1, D), _f32),
        ],
    )(h, part, deg, wr, wn, b, gamma, beta)


def _head_body(xn_ref, w1_ref, b1_ref, w2_ref, b2_ref, w3_ref, b3_ref,
               gsw1_ref, gsb1_ref, gsw2_ref, gsb2_ref,
               ghw1_ref, ghb1_ref, ghw2_ref, ghb2_ref, ghw3_ref, ghb3_ref,
               gout_ref, nout_ref):
    xn = xn_ref[...]
    h1 = lax.dot_general(xn, w1_ref[...], (((2,), (1,)), ((1,), (0,))),
                         preferred_element_type=_f32)
    h1 = jnp.maximum(h1 + b1_ref[...][:, None, :], 0.0)
    h2 = lax.dot_general(h1, w2_ref[...], (((2,), (1,)), ((0,), (0,))),
                         preferred_element_type=_f32)
    h2 = jnp.maximum(h2 + b2_ref[...][:, None, :], 0.0)
    w3 = w3_ref[...][:, :, 0]
    nout_ref[...] = jnp.sum(h2 * w3[:, None, :], axis=2) + b3_ref[...]

    g = jnp.maximum(jnp.sum(xn, axis=1) / NN, 0.0)
    g = jnp.dot(g, gsw1_ref[...], preferred_element_type=_f32) + gsb1_ref[...]
    g = jnp.dot(g, gsw2_ref[...], preferred_element_type=_f32) + gsb2_ref[...]
    g = jnp.maximum(g, 0.0)
    g = jnp.maximum(
        jnp.dot(g, ghw1_ref[...], preferred_element_type=_f32) + ghb1_ref[...], 0.0)
    g = jnp.maximum(
        jnp.dot(g, ghw2_ref[...], preferred_element_type=_f32) + ghb2_ref[...], 0.0)
    gout_ref[...] = (jnp.dot(g, ghw3_ref[...], preferred_element_type=_f32)
                     + ghb3_ref[...])


def _head(xn, nh_W1, nh_b1, nh_W2, nh_b2, nh_W3, nh_b3,
          gs_W1, gs_b1, gs_W2, gs_b2,
          gh_W1, gh_b1, gh_W2, gh_b2, gh_W3, gh_b3):
    return pl.pallas_call(
        _head_body,
        out_shape=[
            jax.ShapeDtypeStruct((B, 1), _f32),
            jax.ShapeDtypeStruct((NN, B), _f32),
        ],
    )(xn, nh_W1, nh_b1, nh_W2, nh_b2, nh_W3, nh_b3,
      gs_W1, gs_b1, gs_W2, gs_b2, gh_W1, gh_b1, gh_W2, gh_b2, gh_W3, gh_b3)


def kernel(x, conv_Wr, conv_Wn, conv_b, bn_gamma, bn_beta,
           gs_W1, gs_b1, gs_W2, gs_b2,
           gh_W1, gh_b1, gh_W2, gh_b2, gh_W3, gh_b3,
           nh_W1, nh_b1, nh_W2, nh_b2, nh_W3, nh_b3,
           edge_index, batch):
    src = edge_index[0]
    dst = edge_index[1]

    padn = EWP - EW
    sfill = (jnp.arange(padn, dtype=jnp.int32) * 997) % N
    dfill = N + (jnp.arange(padn, dtype=jnp.int32) % (NPAD - N))
    srcf = _pad_idx(src, sfill)
    dstf = _pad_idx(dst, dfill)
    dst2 = dstf.reshape(NW, NCH, CH)

    part0_flat, degp, _ = _sc_agg_deg(x, srcf, dstf)
    part0 = part0_flat.reshape(NC, NPAD, D)
    deg = (degp[0] + degp[1])[:N].reshape(N, 1)
    h1 = _conv(x, part0, deg, conv_Wr[0], conv_Wn[0], conv_b[0].reshape(1, D),
               bn_gamma[0:1], bn_beta[0:1])

    part1 = _sc_agg(h1, srcf, dstf)[0].reshape(NC, NPAD, D)
    h2 = _conv(h1, part1, deg, conv_Wr[1], conv_Wn[1], conv_b[1].reshape(1, D),
               bn_gamma[1:2], bn_beta[1:2])

    g_out, n_outT = _head(
        h2.reshape(B, NN, D), nh_W1, nh_b1, nh_W2, nh_b2, nh_W3, nh_b3,
        gs_W1, gs_b1.reshape(1, DS), gs_W2, gs_b2.reshape(1, DS),
        gh_W1, gh_b1.reshape(1, DH), gh_W2, gh_b2.reshape(1, DH),
        gh_W3, gh_b3.reshape(1, 1))
    return jnp.concatenate([g_out, n_outT.T], axis=1)


# speedup vs baseline: 1.1073x; 1.0255x over previous
import functools

import jax
import jax.numpy as jnp
from jax import lax
from jax.experimental import pallas as pl
from jax.experimental.pallas import tpu as pltpu
from jax.experimental.pallas import tpu_sc as plsc

N = 10000
E = 320000
D = 128
B = 100
NN = 100
DS = 64
DH = 64

NC, NS = 2, 16
NW = NC * NS
EW = E // NW

NPAD = 10240
RPS = NPAD // NS
ZR = 64

NCH = 80
CH = 128
EWP = NCH * CH

_f32 = jnp.float32


def _make_sc_agg(with_deg, preload_dst):
    mesh = plsc.VectorSubcoreMesh(core_axis_name="c", subcore_axis_name="s")

    out_type = [jax.ShapeDtypeStruct((NC * NPAD, D), _f32)]
    scratch = [
        pltpu.VMEM((CH,), jnp.int32),
        pltpu.VMEM((CH,), jnp.int32),
        pltpu.VMEM((CH, D), _f32),
        pltpu.VMEM((CH, D), _f32),
        pltpu.VMEM_SHARED((NPAD, D), _f32),
        pltpu.SemaphoreType.DMA,
        pltpu.SemaphoreType.DMA,
        pltpu.SemaphoreType.DMA,
        pltpu.SemaphoreType.DMA,
        pltpu.SemaphoreType.DMA,
    ]
    if preload_dst:
        scratch.insert(2, pltpu.VMEM((NCH, CH), jnp.int32))
    else:
        scratch.insert(2, pltpu.VMEM((CH,), jnp.int32))
        scratch.insert(3, pltpu.VMEM((CH,), jnp.int32))
    if with_deg:
        out_type.append(jax.ShapeDtypeStruct((NC, NPAD), _f32))
        out_type.append(jax.ShapeDtypeStruct((NW, NPAD), _f32))
        scratch += [
            pltpu.VMEM((NPAD,), _f32),
            pltpu.VMEM((RPS,), _f32),
            pltpu.VMEM((RPS,), _f32),
        ]

    @functools.partial(
        pl.kernel, out_type=tuple(out_type), mesh=mesh,
        scratch_types=scratch,
        compiler_params=pltpu.CompilerParams(needs_layout_passes=False))
    def agg(h_hbm, src_hbm, dst_hbm, *rest):
        it = iter(rest)
        out_hbm = next(it)
        if with_deg:
            deg_hbm = next(it)
            stage_hbm = next(it)
        sidx0 = next(it)
        sidx1 = next(it)
        if preload_dst:
            didx2 = next(it)
        else:
            didx0 = next(it)
            didx1 = next(it)
        rows0 = next(it)
        rows1 = next(it)
        acc = next(it)
        gsem0 = next(it)
        gsem1 = next(it)
        ssem0 = next(it)
        ssem1 = next(it)
        isem = next(it)
        if with_deg:
            hist = next(it)
            drow = next(it)
            degv = next(it)

        c = lax.axis_index("c")
        s = lax.axis_index("s")
        wid = s * NC + c
        base = wid * EWP

        if preload_dst:
            pltpu.async_copy(dst_hbm.at[wid], didx2, isem)

        def zrow(j, carry):
            idx = j * 16
            rows0[idx // D, pl.ds(idx % D, 16)] = jnp.zeros((16,), _f32)
            return carry
        lax.fori_loop(0, ZR * D // 16, zrow, 0)
        for k in range(RPS // ZR):
            pltpu.sync_copy(rows0.at[pl.ds(0, ZR)],
                            acc.at[pl.ds(s * RPS + k * ZR, ZR)])

        if with_deg:
            def zhist(j, carry):
                hist[pl.ds(j * 16, 16)] = jnp.zeros((16,), _f32)
                return carry
            lax.fori_loop(0, NPAD // 16, zhist, 0)
            ones = jnp.ones((16,), _f32)

        if preload_dst:
            pltpu.make_async_copy(dst_hbm.at[wid], didx2, isem).wait()
        plsc.subcore_barrier()

        def count(idx_buf):
            if with_deg:
                for k in range(CH // 16):
                    dv = idx_buf[pl.ds(k * 16, 16)]
                    plsc.addupdate_scatter(hist, [dv], ones)

        pltpu.sync_copy(src_hbm.at[pl.ds(base, CH)], sidx0)
        pltpu.async_copy(h_hbm.at[sidx0], rows0, gsem0)
        pltpu.sync_copy(src_hbm.at[pl.ds(base + CH, CH)], sidx1)
        pltpu.async_copy(h_hbm.at[sidx1], rows1, gsem1)

        def body(i, carry):
            j0 = 2 * i
            j1 = j0 + 1
            if preload_dst:
                d0 = didx2.at[j0]
            else:
                pltpu.sync_copy(dst_hbm.at[pl.ds(base + j0 * CH, CH)], didx0)
                d0 = didx0
                count(didx0)
            pltpu.make_async_copy(h_hbm.at[sidx0], rows0, gsem0).wait()
            pltpu.async_copy(rows0, acc.at[d0], ssem0, add=True)
            if preload_dst:
                d1 = didx2.at[j1]
            else:
                pltpu.sync_copy(dst_hbm.at[pl.ds(base + j1 * CH, CH)], didx1)
                d1 = didx1
                count(didx1)
            pltpu.make_async_copy(h_hbm.at[sidx1], rows1, gsem1).wait()
            pltpu.async_copy(rows1, acc.at[d1], ssem1, add=True)
            @pl.when(j0 + 2 < NCH)
            def _():
                pltpu.make_async_copy(rows0, acc.at[d0], ssem0).wait()
                pltpu.sync_copy(
                    src_hbm.at[pl.ds(base + (j0 + 2) * CH, CH)], sidx0)
                pltpu.async_copy(h_hbm.at[sidx0], rows0, gsem0)

            @pl.when(j1 + 2 < NCH)
            def _():
                pltpu.make_async_copy(rows1, acc.at[d1], ssem1).wait()
                pltpu.sync_copy(
                    src_hbm.at[pl.ds(base + (j1 + 2) * CH, CH)], sidx1)
                pltpu.async_copy(h_hbm.at[sidx1], rows1, gsem1)
            return carry

        lax.fori_loop(0, NCH // 2, body, 0)
        if preload_dst:
            dd = didx2.at[0]
        else:
            dd = didx0
        pltpu.make_async_copy(rows0, acc.at[dd], ssem0).wait()
        pltpu.make_async_copy(rows1, acc.at[dd], ssem1).wait()

        if with_deg:
            pltpu.sync_copy(hist, stage_hbm.at[wid])
            plsc.subcore_barrier()
            pltpu.async_copy(
                acc.at[pl.ds(s * RPS, RPS)],
                out_hbm.at[pl.ds(c * NPAD + s * RPS, RPS)], isem)

            def zdeg(j, carry):
                degv[pl.ds(j * 16, 16)] = jnp.zeros((16,), _f32)
                return carry
            lax.fori_loop(0, RPS // 16, zdeg, 0)
            for r in range(NS):
                pltpu.sync_copy(
                    stage_hbm.at[r * NC + c, pl.ds(s * RPS, RPS)], drow)

                def dbody(j, carry):
                    col = j * 16
                    degv[pl.ds(col, 16)] += drow[pl.ds(col, 16)]
                    return carry
                lax.fori_loop(0, RPS // 16, dbody, 0)
            pltpu.sync_copy(degv, deg_hbm.at[c, pl.ds(s * RPS, RPS)])
            pltpu.make_async_copy(
                acc.at[pl.ds(s * RPS, RPS)],
                out_hbm.at[pl.ds(c * NPAD + s * RPS, RPS)], isem).wait()
        else:
            plsc.subcore_barrier()
            pltpu.sync_copy(
                acc.at[pl.ds(s * RPS, RPS)],
                out_hbm.at[pl.ds(c * NPAD + s * RPS, RPS)])

    return agg


_sc_agg_deg = _make_sc_agg(True, False)
_sc_agg = _make_sc_agg(False, False)


def _pad_idx(v, fill):
    per = v.reshape(NW, EW)
    pad = jnp.broadcast_to(fill.reshape(1, EWP - EW), (NW, EWP - EW))
    return jnp.concatenate([per, pad], axis=1).reshape(NW * EWP)

BS = 2000
NBLK = N // BS


def _conv_body(h_ref, part_ref, deg_ref, wr_ref, wn_ref, b_ref,
               gamma_ref, beta_ref, out_ref, hpre_s, sum_s, ssq_s):
    p = pl.program_id(0)
    i = pl.program_id(1)

    @pl.when(p == 0)
    def _():
        agg = part_ref[0] + part_ref[1]
        mean = agg / jnp.maximum(deg_ref[...], 1.0)
        hp = (jnp.dot(h_ref[...], wr_ref[...], preferred_element_type=_f32)
              + jnp.dot(mean, wn_ref[...], preferred_element_type=_f32)
              + b_ref[...])
        hpre_s[pl.ds(i * BS, BS), :] = hp

        @pl.when(i == 0)
        def _():
            sum_s[...] = jnp.zeros_like(sum_s)
            ssq_s[...] = jnp.zeros_like(ssq_s)

        sum_s[...] += jnp.sum(hp, axis=0, keepdims=True)
        ssq_s[...] += jnp.sum(hp * hp, axis=0, keepdims=True)

    @pl.when(p == 1)
    def _():
        mu = sum_s[...] / N
        var = ssq_s[...] / N - mu * mu
        rstd = lax.rsqrt(var + 1e-5)
        hp = hpre_s[pl.ds(i * BS, BS), :]
        hb = (hp - mu) * (rstd * gamma_ref[...]) + beta_ref[...]
        out_ref[...] = jnp.maximum(hb, 0.0)


def _conv(h, part, deg, wr, wn, b, gamma, beta):
    def pin(f):
        return lambda p, i: f(jnp.where(p == 0, i, 0))
    return pl.pallas_call(
        _conv_body,
        grid=(2, NBLK),
        in_specs=[
            pl.BlockSpec((BS, D), pin(lambda i: (i, 0))),
            pl.BlockSpec((NC, BS, D), pin(lambda i: (0, i, 0))),
            pl.BlockSpec((BS, 1), pin(lambda i: (i, 0))),
            pl.BlockSpec((D, D), lambda p, i: (0, 0)),
            pl.BlockSpec((D, D), lambda p, i: (0, 0)),
            pl.BlockSpec((1, D), lambda p, i: (0, 0)),
            pl.BlockSpec((1, D), lambda p, i: (0, 0)),
            pl.BlockSpec((1, D), lambda p, i: (0, 0)),
        ],
        out_specs=pl.BlockSpec((BS, D), lambda p, i: (i, 0)),
        out_shape=jax.ShapeDtypeStruct((N, D), _f32),
        scratch_shapes=[
            pltpu.VMEM((N, D), _f32),
            pltpu.VMEM((1, D), _f32),
            pltpu.VMEM((1, D), _f32),
        ],
    )(h, part, deg, wr, wn, b, gamma, beta)


def _head_body(xn_ref, w1_ref, b1_ref, w2_ref, b2_ref, w3_ref, b3_ref,
               gsw1_ref, gsb1_ref, gsw2_ref, gsb2_ref,
               ghw1_ref, ghb1_ref, ghw2_ref, ghb2_ref, ghw3_ref, ghb3_ref,
               gout_ref, nout_ref):
    xn = xn_ref[...]
    h1 = lax.dot_general(xn, w1_ref[...], (((2,), (1,)), ((1,), (0,))),
                         preferred_element_type=_f32)
    h1 = jnp.maximum(h1 + b1_ref[...][:, None, :], 0.0)
    h2 = lax.dot_general(h1, w2_ref[...], (((2,), (1,)), ((0,), (0,))),
                         preferred_element_type=_f32)
    h2 = jnp.maximum(h2 + b2_ref[...][:, None, :], 0.0)
    w3 = w3_ref[...][:, :, 0]
    nout_ref[...] = jnp.sum(h2 * w3[:, None, :], axis=2) + b3_ref[...]

    g = jnp.maximum(jnp.sum(xn, axis=1) / NN, 0.0)
    g = jnp.dot(g, gsw1_ref[...], preferred_element_type=_f32) + gsb1_ref[...]
    g = jnp.dot(g, gsw2_ref[...], preferred_element_type=_f32) + gsb2_ref[...]
    g = jnp.maximum(g, 0.0)
    g = jnp.maximum(
        jnp.dot(g, ghw1_ref[...], preferred_element_type=_f32) + ghb1_ref[...], 0.0)
    g = jnp.maximum(
        jnp.dot(g, ghw2_ref[...], preferred_element_type=_f32) + ghb2_ref[...], 0.0)
    gout_ref[...] = (jnp.dot(g, ghw3_ref[...], preferred_element_type=_f32)
                     + ghb3_ref[...])


def _head(xn, nh_W1, nh_b1, nh_W2, nh_b2, nh_W3, nh_b3,
          gs_W1, gs_b1, gs_W2, gs_b2,
          gh_W1, gh_b1, gh_W2, gh_b2, gh_W3, gh_b3):
    return pl.pallas_call(
        _head_body,
        out_shape=[
            jax.ShapeDtypeStruct((B, 1), _f32),
            jax.ShapeDtypeStruct((NN, B), _f32),
        ],
    )(xn, nh_W1, nh_b1, nh_W2, nh_b2, nh_W3, nh_b3,
      gs_W1, gs_b1, gs_W2, gs_b2, gh_W1, gh_b1, gh_W2, gh_b2, gh_W3, gh_b3)


def kernel(x, conv_Wr, conv_Wn, conv_b, bn_gamma, bn_beta,
           gs_W1, gs_b1, gs_W2, gs_b2,
           gh_W1, gh_b1, gh_W2, gh_b2, gh_W3, gh_b3,
           nh_W1, nh_b1, nh_W2, nh_b2, nh_W3, nh_b3,
           edge_index, batch):
    src = edge_index[0]
    dst = edge_index[1]

    padn = EWP - EW
    sfill = (jnp.arange(padn, dtype=jnp.int32) * 997) % N
    dfill = N + (jnp.arange(padn, dtype=jnp.int32) % (NPAD - N))
    srcf = _pad_idx(src, sfill)
    dstf = _pad_idx(dst, dfill)
    dst2 = dstf.reshape(NW, NCH, CH)

    part0_flat, degp, _ = _sc_agg_deg(x, srcf, dstf)
    part0 = part0_flat.reshape(NC, NPAD, D)
    deg = (degp[0] + degp[1])[:N].reshape(N, 1)
    h1 = _conv(x, part0, deg, conv_Wr[0], conv_Wn[0], conv_b[0].reshape(1, D),
               bn_gamma[0:1], bn_beta[0:1])

    part1 = _sc_agg(h1, srcf, dstf)[0].reshape(NC, NPAD, D)
    h2 = _conv(h1, part1, deg, conv_Wr[1], conv_Wn[1], conv_b[1].reshape(1, D),
               bn_gamma[1:2], bn_beta[1:2])

    g_out, n_outT = _head(
        h2.reshape(B, NN, D), nh_W1, nh_b1, nh_W2, nh_b2, nh_W3, nh_b3,
        gs_W1, gs_b1.reshape(1, DS), gs_W2, gs_b2.reshape(1, DS),
        gh_W1, gh_b1.reshape(1, DH), gh_W2, gh_b2.reshape(1, DH),
        gh_W3, gh_b3.reshape(1, 1))
    return jnp.concatenate([g_out, n_outT.T], axis=1)
